# Initial kernel scaffold; baseline (speedup 1.0000x reference)
import jax
import jax.numpy as jnp
from jax.experimental import pallas as pl


def kernel(inputs, pos_table):
    raise NotImplementedError("write your pallas kernel here")


# speedup vs baseline: 1.5482x; 1.5482x over previous
import jax
import jax.numpy as jnp
from jax.experimental import pallas as pl


def _add_block(x_ref, p_ref, o_ref):
    o_ref[...] = x_ref[...] + p_ref[...]


def kernel(inputs, pos_table):
    B, S, D = inputs.shape
    SB = 512
    return pl.pallas_call(
        _add_block,
        grid=(S // SB, B),
        in_specs=[
            pl.BlockSpec((1, SB, D), lambda s, b: (b, s, 0)),
            pl.BlockSpec((SB, D), lambda s, b: (s, 0)),
        ],
        out_specs=pl.BlockSpec((1, SB, D), lambda s, b: (b, s, 0)),
        out_shape=jax.ShapeDtypeStruct(inputs.shape, inputs.dtype),
    )(inputs, pos_table)


# speedup vs baseline: 1.5942x; 1.0297x over previous
import jax
import jax.numpy as jnp
from jax.experimental import pallas as pl


def _add_block(x_ref, p_ref, o_ref):
    o_ref[...] = x_ref[...] + p_ref[...]


def kernel(inputs, pos_table):
    B, S, D = inputs.shape
    SB = 1024
    return pl.pallas_call(
        _add_block,
        grid=(S // SB, B),
        in_specs=[
            pl.BlockSpec((1, SB, D), lambda s, b: (b, s, 0)),
            pl.BlockSpec((SB, D), lambda s, b: (s, 0)),
        ],
        out_specs=pl.BlockSpec((1, SB, D), lambda s, b: (b, s, 0)),
        out_shape=jax.ShapeDtypeStruct(inputs.shape, inputs.dtype),
    )(inputs, pos_table)
